# baseline (speedup 1.0000x reference)
import functools

import jax
import jax.numpy as jnp
from jax import lax
from jax.experimental import pallas as pl
from jax.experimental.pallas import tpu as pltpu
from jax.experimental.pallas import tpu_sc as plsc

_R, _C = 128, 100000
_NW = 32
_CPW = 2800
_SCCOLS = _NW * _CPW
_CHW = 280
_NCH = _CPW // _CHW
_TBLK = 800
_TGRID = (_C - _SCCOLS) // _TBLK
_TOFF = _SCCOLS // _TBLK
_NEG = -3.4e38
_BIG = 2**30

def _sc_body(tout_hbm, val_hbm, idx_hbm, buf0, buf1, valv, idxv, sem0, sem1):
    cid = lax.axis_index("c")
    sid = lax.axis_index("s")
    wid = cid * 16 + sid
    col0 = wid * _CPW

    bufs = (buf0, buf1)
    sems = (sem0, sem1)

    def cstart(j):
        return pl.multiple_of(col0 + j * _CHW, 8)

    def scan_cols(buf, pos0, carry, ncols):
        def body(i, c):
            st = list(c[:-1])
            pos = c[-1]
            for u in range(2):
                col = i * 2 + u
                p = pos + u
                for g in range(8):
                    v = buf[col, pl.ds(g * 16, 16)]
                    upd = v > st[2 * g]
                    st[2 * g] = jnp.where(upd, v, st[2 * g])
                    st[2 * g + 1] = jnp.where(upd, p, st[2 * g + 1])
            return tuple(st) + (pos + 2,)

        out = lax.fori_loop(0, ncols // 2, body, tuple(carry) + (pos0,))
        return out[:-1]

    pltpu.async_copy(tout_hbm.at[pl.ds(cstart(0), _CHW)], bufs[0], sems[0])
    pltpu.async_copy(tout_hbm.at[pl.ds(cstart(1), _CHW)], bufs[1], sems[1])

    carry = []
    for g in range(8):
        carry += [jnp.full((16,), _NEG, jnp.float32),
                  jnp.zeros((16,), jnp.int32)]
    carry = tuple(carry)

    def outer(jp, carry):
        for b in range(2):
            j = jp * 2 + b
            pltpu.make_async_copy(
                tout_hbm.at[pl.ds(cstart(0), _CHW)], bufs[b],
                sems[b]).wait()
            c2 = scan_cols(bufs[b], jnp.full((16,), 0, jnp.int32)
                           + (col0 + j * _CHW), carry, _CHW)

            @pl.when(j + 2 < _NCH)
            def _():
                pltpu.async_copy(
                    tout_hbm.at[pl.ds(cstart(j + 2), _CHW)], bufs[b],
                    sems[b])

            carry = c2
        return carry

    carry = lax.fori_loop(0, _NCH // 2, outer, carry)

    for g in range(8):
        valv[pl.ds(g * 16, 16)] = carry[2 * g]
        idxv[pl.ds(g * 16, 16)] = carry[2 * g + 1]
    pltpu.sync_copy(valv, val_hbm.at[pl.ds(wid * 128, 128)])
    pltpu.sync_copy(idxv, idx_hbm.at[pl.ds(wid * 128, 128)])


_sc_argmax = functools.partial(
    pl.kernel,
    out_type=[
        jax.ShapeDtypeStruct((_NW * 128,), jnp.float32),
        jax.ShapeDtypeStruct((_NW * 128,), jnp.int32),
    ],
    mesh=plsc.VectorSubcoreMesh(core_axis_name="c", subcore_axis_name="s"),
    scratch_types=[
        pltpu.VMEM((_CHW, 128), jnp.float32),
        pltpu.VMEM((_CHW, 128), jnp.float32),
        pltpu.VMEM((128,), jnp.float32),
        pltpu.VMEM((128,), jnp.int32),
        pltpu.SemaphoreType.DMA,
        pltpu.SemaphoreType.DMA,
    ],
)(_sc_body)


_SBLK = 5000


def _tc_sums_body(l_ref, s_ref, sll_ref):
    l = l_ref[...]
    ps = jnp.sum(l, axis=0, keepdims=True)
    psll = jnp.sum(l * jnp.log(l), axis=0, keepdims=True)

    @pl.when(pl.program_id(0) == 0)
    def _():
        s_ref[...] = jnp.zeros_like(s_ref)
        sll_ref[...] = jnp.zeros_like(sll_ref)

    s_ref[...] += ps
    sll_ref[...] += psll


def _tc_argmax_body(o_ref, tval_ref, tidx_ref, rm_s, ri_s):
    i = pl.program_id(0)
    o = o_ref[...]
    base = _SCCOLS + i * _TBLK
    bm = jnp.max(o, axis=0, keepdims=True)
    iota = lax.broadcasted_iota(jnp.int32, (_TBLK, 128), 0) + base
    bi = jnp.min(jnp.where(o == bm, iota, _BIG), axis=0, keepdims=True)

    @pl.when(i == 0)
    def _():
        rm_s[...] = jnp.full_like(rm_s, _NEG)
        ri_s[...] = jnp.zeros_like(ri_s)

    upd = bm > rm_s[...]
    rm_s[...] = jnp.where(upd, bm, rm_s[...])
    ri_s[...] = jnp.where(upd, bi, ri_s[...])

    @pl.when(i == _TGRID - 1)
    def _():
        tval_ref[...] = rm_s[...]
        tidx_ref[...] = ri_s[...]


def _merge_body(val_ref, idx_ref, tval_ref, tidx_ref, s_ref, sll_ref,
                ri_ref, ent_ref, logs_ref):
    val = val_ref[...]
    idx = idx_ref[...]
    tval = tval_ref[...]
    tidx = tidx_ref[...]
    m = jnp.maximum(jnp.max(val, axis=0, keepdims=True), tval)
    ri = jnp.min(jnp.where(val == m, idx, _BIG), axis=0, keepdims=True)
    ri = jnp.minimum(ri, jnp.where(tval == m, tidx, _BIG))
    s = s_ref[...]
    logs = jnp.log(s)
    ri_ref[...] = ri
    ent_ref[...] = logs - sll_ref[...] / s
    logs_ref[...] = logs


def _gather_body(ri_smem, ri_vmem, tl_hbm, logs_ref, alp_ref, gbuf, sem):
    copies = []
    for r in range(_R):
        base = pl.multiple_of((ri_smem[0, r] >> 3) << 3, 8)
        copies.append(pltpu.make_async_copy(
            tl_hbm.at[pl.ds(base, 8)], gbuf.at[r], sem))
        copies[-1].start()
    for c in copies:
        c.wait()
    g = gbuf[...]
    sub = ri_vmem[...] & 7
    row_i = lax.broadcasted_iota(jnp.int32, (_R, 8, 128), 0)
    sub_i = lax.broadcasted_iota(jnp.int32, (_R, 8, 128), 1)
    lane_i = lax.broadcasted_iota(jnp.int32, (_R, 8, 128), 2)
    pick = (lane_i == row_i) & (sub_i == sub.reshape(_R)[:, None, None])
    lsel = jnp.sum(jnp.where(pick, g, 0.0), axis=(1, 2))
    alp_ref[...] = jnp.log(lsel).reshape(1, _R) - logs_ref[...]


def kernel(logits, outputs):
    tl = logits.T
    tout = outputs.T
    val, idx = _sc_argmax(tout)
    s, sll = pl.pallas_call(
        _tc_sums_body,
        grid=(_C // _SBLK,),
        in_specs=[pl.BlockSpec((_SBLK, 128), lambda i: (i, 0))],
        out_specs=[
            pl.BlockSpec((1, 128), lambda i: (0, 0)),
            pl.BlockSpec((1, 128), lambda i: (0, 0)),
        ],
        out_shape=[
            jax.ShapeDtypeStruct((1, 128), jnp.float32),
            jax.ShapeDtypeStruct((1, 128), jnp.float32),
        ],
    )(tl)
    tval, tidx = pl.pallas_call(
        _tc_argmax_body,
        grid=(_TGRID,),
        in_specs=[pl.BlockSpec((_TBLK, 128), lambda i: (i + _TOFF, 0))],
        out_specs=[
            pl.BlockSpec((1, 128), lambda i: (0, 0)),
            pl.BlockSpec((1, 128), lambda i: (0, 0)),
        ],
        out_shape=[
            jax.ShapeDtypeStruct((1, 128), jnp.float32),
            jax.ShapeDtypeStruct((1, 128), jnp.int32),
        ],
        scratch_shapes=[
            pltpu.VMEM((1, 128), jnp.float32),
            pltpu.VMEM((1, 128), jnp.int32),
        ],
    )(tout)
    ri, ent, logs = pl.pallas_call(
        _merge_body,
        out_shape=[
            jax.ShapeDtypeStruct((1, 128), jnp.int32),
            jax.ShapeDtypeStruct((1, 128), jnp.float32),
            jax.ShapeDtypeStruct((1, 128), jnp.float32),
        ],
    )(val.reshape(_NW, 128), idx.reshape(_NW, 128), tval, tidx, s, sll)
    alp = pl.pallas_call(
        _gather_body,
        in_specs=[
            pl.BlockSpec(memory_space=pltpu.SMEM),
            pl.BlockSpec(memory_space=pltpu.VMEM),
            pl.BlockSpec(memory_space=pltpu.MemorySpace.HBM),
            pl.BlockSpec(memory_space=pltpu.VMEM),
        ],
        out_specs=pl.BlockSpec(memory_space=pltpu.VMEM),
        out_shape=jax.ShapeDtypeStruct((1, _R), jnp.float32),
        scratch_shapes=[
            pltpu.VMEM((_R, 8, 128), jnp.float32),
            pltpu.SemaphoreType.DMA,
        ],
    )(ri, ri, tl, logs)
    return (alp.reshape(_R), ent.reshape(_R))


# speedup vs baseline: 1.0393x; 1.0393x over previous
import functools

import jax
import jax.numpy as jnp
from jax import lax
from jax.experimental import pallas as pl
from jax.experimental.pallas import tpu as pltpu
from jax.experimental.pallas import tpu_sc as plsc

_R, _C = 128, 100000
_NW = 32
_CPW = 2800
_SCCOLS = _NW * _CPW
_CHW = 280
_NCH = _CPW // _CHW
_TBLK = 800
_TGRID = (_C - _SCCOLS) // _TBLK
_TOFF = _SCCOLS // _TBLK
_NEG = -3.4e38
_BIG = 2**30

def _sc_body(tout_hbm, val_hbm, idx_hbm, buf0, buf1, valv, idxv, sem0, sem1):
    cid = lax.axis_index("c")
    sid = lax.axis_index("s")
    wid = cid * 16 + sid
    col0 = wid * _CPW

    bufs = (buf0, buf1)
    sems = (sem0, sem1)

    def cstart(j):
        return pl.multiple_of(col0 + j * _CHW, 8)

    def scan_cols(buf, pos0, carry, ncols):
        def body(i, c):
            st = list(c[:-1])
            pos = c[-1]
            for u in range(2):
                col = i * 2 + u
                p = pos + u
                for g in range(8):
                    v = buf[col, pl.ds(g * 16, 16)]
                    upd = v > st[2 * g]
                    st[2 * g] = jnp.where(upd, v, st[2 * g])
                    st[2 * g + 1] = jnp.where(upd, p, st[2 * g + 1])
            return tuple(st) + (pos + 2,)

        out = lax.fori_loop(0, ncols // 2, body, tuple(carry) + (pos0,))
        return out[:-1]

    pltpu.async_copy(tout_hbm.at[pl.ds(cstart(0), _CHW)], bufs[0], sems[0])
    pltpu.async_copy(tout_hbm.at[pl.ds(cstart(1), _CHW)], bufs[1], sems[1])

    carry = []
    for g in range(8):
        carry += [jnp.full((16,), _NEG, jnp.float32),
                  jnp.zeros((16,), jnp.int32)]
    carry = tuple(carry)

    def outer(jp, carry):
        for b in range(2):
            j = jp * 2 + b
            pltpu.make_async_copy(
                tout_hbm.at[pl.ds(cstart(0), _CHW)], bufs[b],
                sems[b]).wait()
            c2 = scan_cols(bufs[b], jnp.full((16,), 0, jnp.int32)
                           + (col0 + j * _CHW), carry, _CHW)

            @pl.when(j + 2 < _NCH)
            def _():
                pltpu.async_copy(
                    tout_hbm.at[pl.ds(cstart(j + 2), _CHW)], bufs[b],
                    sems[b])

            carry = c2
        return carry

    carry = lax.fori_loop(0, _NCH // 2, outer, carry)

    for g in range(8):
        valv[pl.ds(g * 16, 16)] = carry[2 * g]
        idxv[pl.ds(g * 16, 16)] = carry[2 * g + 1]
    pltpu.sync_copy(valv, val_hbm.at[pl.ds(wid * 128, 128)])
    pltpu.sync_copy(idxv, idx_hbm.at[pl.ds(wid * 128, 128)])


_sc_argmax = functools.partial(
    pl.kernel,
    out_type=[
        jax.ShapeDtypeStruct((_NW * 128,), jnp.float32),
        jax.ShapeDtypeStruct((_NW * 128,), jnp.int32),
    ],
    mesh=plsc.VectorSubcoreMesh(core_axis_name="c", subcore_axis_name="s"),
    scratch_types=[
        pltpu.VMEM((_CHW, 128), jnp.float32),
        pltpu.VMEM((_CHW, 128), jnp.float32),
        pltpu.VMEM((128,), jnp.float32),
        pltpu.VMEM((128,), jnp.int32),
        pltpu.SemaphoreType.DMA,
        pltpu.SemaphoreType.DMA,
    ],
)(_sc_body)


_SBLK = 10000


def _tc_sums_body(l_ref, s_ref, sll_ref):
    l = l_ref[...]
    ps = jnp.sum(l, axis=0, keepdims=True)
    psll = jnp.sum(l * jnp.log(l), axis=0, keepdims=True)

    @pl.when(pl.program_id(0) == 0)
    def _():
        s_ref[...] = jnp.zeros_like(s_ref)
        sll_ref[...] = jnp.zeros_like(sll_ref)

    s_ref[...] += ps
    sll_ref[...] += psll


def _tc_argmax_body(o_ref, tval_ref, tidx_ref, rm_s, ri_s):
    i = pl.program_id(0)
    o = o_ref[...]
    base = _SCCOLS + i * _TBLK
    bm = jnp.max(o, axis=0, keepdims=True)
    iota = lax.broadcasted_iota(jnp.int32, (_TBLK, 128), 0) + base
    bi = jnp.min(jnp.where(o == bm, iota, _BIG), axis=0, keepdims=True)

    @pl.when(i == 0)
    def _():
        rm_s[...] = jnp.full_like(rm_s, _NEG)
        ri_s[...] = jnp.zeros_like(ri_s)

    upd = bm > rm_s[...]
    rm_s[...] = jnp.where(upd, bm, rm_s[...])
    ri_s[...] = jnp.where(upd, bi, ri_s[...])

    @pl.when(i == _TGRID - 1)
    def _():
        tval_ref[...] = rm_s[...]
        tidx_ref[...] = ri_s[...]


def _merge_body(val_ref, idx_ref, tval_ref, tidx_ref, s_ref, sll_ref,
                ri_ref, ent_ref, logs_ref):
    val = val_ref[...]
    idx = idx_ref[...]
    tval = tval_ref[...]
    tidx = tidx_ref[...]
    m = jnp.maximum(jnp.max(val, axis=0, keepdims=True), tval)
    ri = jnp.min(jnp.where(val == m, idx, _BIG), axis=0, keepdims=True)
    ri = jnp.minimum(ri, jnp.where(tval == m, tidx, _BIG))
    s = s_ref[...]
    logs = jnp.log(s)
    ri_ref[...] = ri
    ent_ref[...] = logs - sll_ref[...] / s
    logs_ref[...] = logs


def _gather_body(ri_smem, ri_vmem, tl_hbm, logs_ref, alp_ref, gbuf, sem):
    copies = []
    for r in range(_R):
        base = pl.multiple_of((ri_smem[0, r] >> 3) << 3, 8)
        copies.append(pltpu.make_async_copy(
            tl_hbm.at[pl.ds(base, 8)], gbuf.at[r], sem))
        copies[-1].start()
    for c in copies:
        c.wait()
    g = gbuf[...]
    sub = ri_vmem[...] & 7
    row_i = lax.broadcasted_iota(jnp.int32, (_R, 8, 128), 0)
    sub_i = lax.broadcasted_iota(jnp.int32, (_R, 8, 128), 1)
    lane_i = lax.broadcasted_iota(jnp.int32, (_R, 8, 128), 2)
    pick = (lane_i == row_i) & (sub_i == sub.reshape(_R)[:, None, None])
    lsel = jnp.sum(jnp.where(pick, g, 0.0), axis=(1, 2))
    alp_ref[...] = jnp.log(lsel).reshape(1, _R) - logs_ref[...]


def kernel(logits, outputs):
    tl = logits.T
    tout = outputs.T
    val, idx = _sc_argmax(tout)
    s, sll = pl.pallas_call(
        _tc_sums_body,
        grid=(_C // _SBLK,),
        in_specs=[pl.BlockSpec((_SBLK, 128), lambda i: (i, 0))],
        out_specs=[
            pl.BlockSpec((1, 128), lambda i: (0, 0)),
            pl.BlockSpec((1, 128), lambda i: (0, 0)),
        ],
        out_shape=[
            jax.ShapeDtypeStruct((1, 128), jnp.float32),
            jax.ShapeDtypeStruct((1, 128), jnp.float32),
        ],
    )(tl)
    tval, tidx = pl.pallas_call(
        _tc_argmax_body,
        grid=(_TGRID,),
        in_specs=[pl.BlockSpec((_TBLK, 128), lambda i: (i + _TOFF, 0))],
        out_specs=[
            pl.BlockSpec((1, 128), lambda i: (0, 0)),
            pl.BlockSpec((1, 128), lambda i: (0, 0)),
        ],
        out_shape=[
            jax.ShapeDtypeStruct((1, 128), jnp.float32),
            jax.ShapeDtypeStruct((1, 128), jnp.int32),
        ],
        scratch_shapes=[
            pltpu.VMEM((1, 128), jnp.float32),
            pltpu.VMEM((1, 128), jnp.int32),
        ],
    )(tout)
    ri, ent, logs = pl.pallas_call(
        _merge_body,
        out_shape=[
            jax.ShapeDtypeStruct((1, 128), jnp.int32),
            jax.ShapeDtypeStruct((1, 128), jnp.float32),
            jax.ShapeDtypeStruct((1, 128), jnp.float32),
        ],
    )(val.reshape(_NW, 128), idx.reshape(_NW, 128), tval, tidx, s, sll)
    alp = pl.pallas_call(
        _gather_body,
        in_specs=[
            pl.BlockSpec(memory_space=pltpu.SMEM),
            pl.BlockSpec(memory_space=pltpu.VMEM),
            pl.BlockSpec(memory_space=pltpu.MemorySpace.HBM),
            pl.BlockSpec(memory_space=pltpu.VMEM),
        ],
        out_specs=pl.BlockSpec(memory_space=pltpu.VMEM),
        out_shape=jax.ShapeDtypeStruct((1, _R), jnp.float32),
        scratch_shapes=[
            pltpu.VMEM((_R, 8, 128), jnp.float32),
            pltpu.SemaphoreType.DMA,
        ],
    )(ri, ri, tl, logs)
    return (alp.reshape(_R), ent.reshape(_R))


# speedup vs baseline: 1.0884x; 1.0472x over previous
import functools

import jax
import jax.numpy as jnp
from jax import lax
from jax.experimental import pallas as pl
from jax.experimental.pallas import tpu as pltpu
from jax.experimental.pallas import tpu_sc as plsc

_R, _C = 128, 100000
_NW = 32
_CPW = 3000
_SCCOLS = _NW * _CPW
_CHW = 200
_NCH = _CPW // _CHW
_TBLK = 4000
_TGRID = (_C - _SCCOLS) // _TBLK
_TOFF = _SCCOLS // _TBLK
_NEG = -3.4e38
_BIG = 2**30

def _sc_body(tout_hbm, val_hbm, idx_hbm, buf0, buf1, valv, idxv, sem0, sem1):
    cid = lax.axis_index("c")
    sid = lax.axis_index("s")
    wid = cid * 16 + sid
    col0 = wid * _CPW

    bufs = (buf0, buf1)
    sems = (sem0, sem1)

    def cstart(j):
        return pl.multiple_of(col0 + j * _CHW, 8)

    def scan_cols(buf, pos0, carry, ncols):
        def body(i, c):
            st = list(c[:-1])
            pos = c[-1]
            for u in range(2):
                col = i * 2 + u
                p = pos + u
                for g in range(8):
                    v = buf[col, pl.ds(g * 16, 16)]
                    upd = v > st[2 * g]
                    st[2 * g] = jnp.where(upd, v, st[2 * g])
                    st[2 * g + 1] = jnp.where(upd, p, st[2 * g + 1])
            return tuple(st) + (pos + 2,)

        out = lax.fori_loop(0, ncols // 2, body, tuple(carry) + (pos0,))
        return out[:-1]

    pltpu.async_copy(tout_hbm.at[pl.ds(cstart(0), _CHW)], bufs[0], sems[0])
    pltpu.async_copy(tout_hbm.at[pl.ds(cstart(1), _CHW)], bufs[1], sems[1])

    carry = []
    for g in range(8):
        carry += [jnp.full((16,), _NEG, jnp.float32),
                  jnp.zeros((16,), jnp.int32)]
    carry = tuple(carry)

    def outer(jp, carry):
        for b in range(2):
            j = jp * 2 + b
            pltpu.make_async_copy(
                tout_hbm.at[pl.ds(cstart(0), _CHW)], bufs[b],
                sems[b]).wait()
            c2 = scan_cols(bufs[b], jnp.full((16,), 0, jnp.int32)
                           + (col0 + j * _CHW), carry, _CHW)

            @pl.when(j + 2 < _NCH)
            def _():
                pltpu.async_copy(
                    tout_hbm.at[pl.ds(cstart(j + 2), _CHW)], bufs[b],
                    sems[b])

            carry = c2
        return carry

    carry = lax.fori_loop(0, _NCH // 2, outer, carry)
    if _NCH % 2:
        j = _NCH - 1
        pltpu.make_async_copy(
            tout_hbm.at[pl.ds(cstart(0), _CHW)], bufs[0], sems[0]).wait()
        carry = scan_cols(bufs[0], jnp.full((16,), 0, jnp.int32)
                          + (col0 + j * _CHW), carry, _CHW)

    for g in range(8):
        valv[pl.ds(g * 16, 16)] = carry[2 * g]
        idxv[pl.ds(g * 16, 16)] = carry[2 * g + 1]
    pltpu.sync_copy(valv, val_hbm.at[pl.ds(wid * 128, 128)])
    pltpu.sync_copy(idxv, idx_hbm.at[pl.ds(wid * 128, 128)])


_sc_argmax = functools.partial(
    pl.kernel,
    out_type=[
        jax.ShapeDtypeStruct((_NW * 128,), jnp.float32),
        jax.ShapeDtypeStruct((_NW * 128,), jnp.int32),
    ],
    mesh=plsc.VectorSubcoreMesh(core_axis_name="c", subcore_axis_name="s"),
    scratch_types=[
        pltpu.VMEM((_CHW, 128), jnp.float32),
        pltpu.VMEM((_CHW, 128), jnp.float32),
        pltpu.VMEM((128,), jnp.float32),
        pltpu.VMEM((128,), jnp.int32),
        pltpu.SemaphoreType.DMA,
        pltpu.SemaphoreType.DMA,
    ],
)(_sc_body)


_SBLK = 10000


def _tc_sums_body(l_ref, s_ref, sll_ref):
    l = l_ref[...]
    ps = jnp.sum(l, axis=0, keepdims=True)
    psll = jnp.sum(l * jnp.log(l), axis=0, keepdims=True)

    @pl.when(pl.program_id(0) == 0)
    def _():
        s_ref[...] = jnp.zeros_like(s_ref)
        sll_ref[...] = jnp.zeros_like(sll_ref)

    s_ref[...] += ps
    sll_ref[...] += psll


def _tc_argmax_body(o_ref, tval_ref, tidx_ref, rm_s, ri_s):
    i = pl.program_id(0)
    o = o_ref[...]
    base = _SCCOLS + i * _TBLK
    bm = jnp.max(o, axis=0, keepdims=True)
    iota = lax.broadcasted_iota(jnp.int32, (_TBLK, 128), 0) + base
    bi = jnp.min(jnp.where(o == bm, iota, _BIG), axis=0, keepdims=True)

    @pl.when(i == 0)
    def _():
        rm_s[...] = jnp.full_like(rm_s, _NEG)
        ri_s[...] = jnp.zeros_like(ri_s)

    upd = bm > rm_s[...]
    rm_s[...] = jnp.where(upd, bm, rm_s[...])
    ri_s[...] = jnp.where(upd, bi, ri_s[...])

    @pl.when(i == _TGRID - 1)
    def _():
        tval_ref[...] = rm_s[...]
        tidx_ref[...] = ri_s[...]


def _merge_body(val_ref, idx_ref, tval_ref, tidx_ref, s_ref, sll_ref,
                ri_ref, ent_ref, logs_ref):
    val = val_ref[...]
    idx = idx_ref[...]
    tval = tval_ref[...]
    tidx = tidx_ref[...]
    m = jnp.maximum(jnp.max(val, axis=0, keepdims=True), tval)
    ri = jnp.min(jnp.where(val == m, idx, _BIG), axis=0, keepdims=True)
    ri = jnp.minimum(ri, jnp.where(tval == m, tidx, _BIG))
    s = s_ref[...]
    logs = jnp.log(s)
    ri_ref[...] = ri
    ent_ref[...] = logs - sll_ref[...] / s
    logs_ref[...] = logs


def _gather_body(ri_smem, ri_vmem, tl_hbm, logs_ref, alp_ref, gbuf, sem):
    copies = []
    for r in range(_R):
        base = pl.multiple_of((ri_smem[0, r] >> 3) << 3, 8)
        copies.append(pltpu.make_async_copy(
            tl_hbm.at[pl.ds(base, 8)], gbuf.at[r], sem))
        copies[-1].start()
    for c in copies:
        c.wait()
    g = gbuf[...]
    sub = ri_vmem[...] & 7
    row_i = lax.broadcasted_iota(jnp.int32, (_R, 8, 128), 0)
    sub_i = lax.broadcasted_iota(jnp.int32, (_R, 8, 128), 1)
    lane_i = lax.broadcasted_iota(jnp.int32, (_R, 8, 128), 2)
    pick = (lane_i == row_i) & (sub_i == sub.reshape(_R)[:, None, None])
    lsel = jnp.sum(jnp.where(pick, g, 0.0), axis=(1, 2))
    alp_ref[...] = jnp.log(lsel).reshape(1, _R) - logs_ref[...]


def kernel(logits, outputs):
    tl = logits.T
    tout = outputs.T
    val, idx = _sc_argmax(tout)
    s, sll = pl.pallas_call(
        _tc_sums_body,
        grid=(_C // _SBLK,),
        in_specs=[pl.BlockSpec((_SBLK, 128), lambda i: (i, 0))],
        out_specs=[
            pl.BlockSpec((1, 128), lambda i: (0, 0)),
            pl.BlockSpec((1, 128), lambda i: (0, 0)),
        ],
        out_shape=[
            jax.ShapeDtypeStruct((1, 128), jnp.float32),
            jax.ShapeDtypeStruct((1, 128), jnp.float32),
        ],
    )(tl)
    tval, tidx = pl.pallas_call(
        _tc_argmax_body,
        grid=(_TGRID,),
        in_specs=[pl.BlockSpec((_TBLK, 128), lambda i: (i + _TOFF, 0))],
        out_specs=[
            pl.BlockSpec((1, 128), lambda i: (0, 0)),
            pl.BlockSpec((1, 128), lambda i: (0, 0)),
        ],
        out_shape=[
            jax.ShapeDtypeStruct((1, 128), jnp.float32),
            jax.ShapeDtypeStruct((1, 128), jnp.int32),
        ],
        scratch_shapes=[
            pltpu.VMEM((1, 128), jnp.float32),
            pltpu.VMEM((1, 128), jnp.int32),
        ],
    )(tout)
    ri, ent, logs = pl.pallas_call(
        _merge_body,
        out_shape=[
            jax.ShapeDtypeStruct((1, 128), jnp.int32),
            jax.ShapeDtypeStruct((1, 128), jnp.float32),
            jax.ShapeDtypeStruct((1, 128), jnp.float32),
        ],
    )(val.reshape(_NW, 128), idx.reshape(_NW, 128), tval, tidx, s, sll)
    alp = pl.pallas_call(
        _gather_body,
        in_specs=[
            pl.BlockSpec(memory_space=pltpu.SMEM),
            pl.BlockSpec(memory_space=pltpu.VMEM),
            pl.BlockSpec(memory_space=pltpu.MemorySpace.HBM),
            pl.BlockSpec(memory_space=pltpu.VMEM),
        ],
        out_specs=pl.BlockSpec(memory_space=pltpu.VMEM),
        out_shape=jax.ShapeDtypeStruct((1, _R), jnp.float32),
        scratch_shapes=[
            pltpu.VMEM((_R, 8, 128), jnp.float32),
            pltpu.SemaphoreType.DMA,
        ],
    )(ri, ri, tl, logs)
    return (alp.reshape(_R), ent.reshape(_R))
